# Initial kernel scaffold; baseline (speedup 1.0000x reference)
import jax
import jax.numpy as jnp
from jax.experimental import pallas as pl


def kernel(input_ids, attention_mask, codewords, text_embeddings, position_bias, sem_table, pos_table):
    raise NotImplementedError("write your pallas kernel here")


# speedup vs baseline: 1.4370x; 1.4370x over previous
import jax
import jax.numpy as jnp
from jax import lax
from jax.experimental import pallas as pl
from jax.experimental.pallas import tpu as pltpu
from jax.experimental.pallas import tpu_sc as plsc

B, S, D = 2, 2048, 768
EMB = 1024
LANES = 16
NROWS = B * S
TRASH = NROWS
PAD_ROWS = NROWS + 8
ROWS_PER_W = 128
CHUNK = 64
NSUB = ROWS_PER_W // CHUNK
W_PER_B = 16


def _splat_last(vec):
    idx = jnp.full((LANES, 1), LANES - 1, jnp.int32)
    dnums = lax.GatherDimensionNumbers(
        offset_dims=(), collapsed_slice_dims=(0,), start_index_map=(0,))
    return lax.gather(vec, idx, dnums, (1,),
                      mode=lax.GatherScatterMode.PROMISE_IN_BOUNDS)


def _sc_body(mask_hbm, cw_hbm, text_hbm, sem_hbm, pos_hbm,
             out_hbm, mout_hbm,
             mask_v, cw_v, semidx_v, relidx_v, dstidx_v, rows_v, pos_v,
             dma_sem):
    wid = lax.axis_index("s") * 2 + lax.axis_index("c")
    b = wid // W_PER_B
    w_in_b = wid % W_PER_B
    s0w = w_in_b * ROWS_PER_W

    pltpu.sync_copy(mask_hbm.at[b], mask_v)
    pltpu.sync_copy(cw_hbm.at[b], cw_v)

    def _len_body(i, acc):
        return acc + mask_v[pl.ds(i * LANES, LANES)]

    acc = lax.fori_loop(0, S // LANES, _len_body,
                        jnp.zeros((LANES,), jnp.int32))
    ln_vec = _splat_last(plsc.cumsum(acc))
    full_vec = ln_vec >= (S - EMB)

    @pl.when(w_in_b == 0)
    def _write_mask():
        def _m(i, carry):
            s_vec = lax.iota(jnp.int32, LANES) + i * LANES
            in_ctx = (s_vec >= ln_vec) & (s_vec < ln_vec + EMB)
            old = mask_v[pl.ds(i * LANES, LANES)]
            mask_v[pl.ds(i * LANES, LANES)] = jnp.where(
                full_vec | in_ctx, jnp.ones((LANES,), jnp.int32), old)
            return carry

        lax.fori_loop(0, S // LANES, _m, 0)
        pltpu.sync_copy(mask_v, mout_hbm.at[b])

    for c in range(NSUB):
        s0 = s0w + c * CHUNK
        gbase = b * S + s0

        any_acc = jnp.zeros((LANES,), jnp.bool_)
        all_acc = jnp.ones((LANES,), jnp.bool_)
        for k in range(CHUNK // LANES):
            s_vec = lax.iota(jnp.int32, LANES) + (s0 + k * LANES)
            rel = s_vec - ln_vec
            relc = jnp.clip(rel, 0, EMB - 1)
            semidx_v[pl.ds(k * LANES, LANES)] = plsc.load_gather(cw_v, [relc])
            relidx_v[pl.ds(k * LANES, LANES)] = relc
            in_c = (rel >= 0) & (rel < EMB)
            any_acc = any_acc | in_c
            all_acc = all_acc & in_c
            dstidx_v[pl.ds(k * LANES, LANES)] = jnp.where(
                in_c, s_vec + b * S, jnp.full((LANES,), TRASH, jnp.int32))
        has_ctx = jnp.any(any_acc)
        full_in = jnp.all(all_acc)

        @pl.when(jnp.logical_not(full_in))
        def _copy_text():
            pltpu.sync_copy(text_hbm.at[pl.ds(gbase, CHUNK)], pos_v)
            pltpu.sync_copy(pos_v, out_hbm.at[pl.ds(gbase, CHUNK)])

        @pl.when(has_ctx)
        def _ctx():
            pltpu.async_copy(sem_hbm.at[semidx_v], rows_v, dma_sem).wait()
            pltpu.async_copy(pos_hbm.at[relidx_v], pos_v, dma_sem).wait()

            def _add(r, carry):
                for k in range(D // LANES):
                    sl = pl.ds(k * LANES, LANES)
                    rows_v[r, sl] = rows_v[r, sl] + pos_v[r, sl]
                return carry

            lax.fori_loop(0, CHUNK, _add, 0)
            pltpu.async_copy(rows_v, out_hbm.at[dstidx_v], dma_sem).wait()


@jax.jit
def _run(attention_mask, codewords, text2d, sem_table, pos_table):
    mesh = plsc.VectorSubcoreMesh(core_axis_name="c", subcore_axis_name="s")
    call = pl.kernel(
        _sc_body,
        out_type=(
            jax.ShapeDtypeStruct((PAD_ROWS, D), jnp.float32),
            jax.ShapeDtypeStruct((B, S), jnp.int32),
        ),
        mesh=mesh,
        scratch_types=[
            pltpu.VMEM((S,), jnp.int32),
            pltpu.VMEM((EMB,), jnp.int32),
            pltpu.VMEM((CHUNK,), jnp.int32),
            pltpu.VMEM((CHUNK,), jnp.int32),
            pltpu.VMEM((CHUNK,), jnp.int32),
            pltpu.VMEM((CHUNK, D), jnp.float32),
            pltpu.VMEM((CHUNK, D), jnp.float32),
            pltpu.SemaphoreType.DMA,
        ],
        compiler_params=pltpu.CompilerParams(needs_layout_passes=False),
    )
    return call(attention_mask, codewords, text2d, sem_table, pos_table)


def kernel(input_ids, attention_mask, codewords, text_embeddings,
           position_bias, sem_table, pos_table):
    padded, mask_out = _run(attention_mask, codewords,
                            text_embeddings.reshape(NROWS, D),
                            sem_table, pos_table)
    inputs_embeds = padded[:NROWS].reshape(B, S, D)
    return inputs_embeds, mask_out, position_bias
